# baseline (speedup 1.0000x reference)
import functools

import jax
import jax.numpy as jnp
from jax import lax
from jax.experimental import pallas as pl
from jax.experimental.pallas import tpu as pltpu
from jax.experimental.pallas import tpu_sc as plsc

BATCH = 16384
EMBED = 16
NUM_ROWS = 1000000
NW = 32
RANGE = NUM_ROWS // NW
NCH = 16
CHUNK = 2048
TAIL_START = (NUM_ROWS // 128) * 128
TAIL = NUM_ROWS - TAIL_START
MAX_START = ((NUM_ROWS - CHUNK) // 128) * 128
NSLOT = 4
DUMMY = BATCH


@functools.lru_cache(maxsize=None)
def _make_sc_sweep():
    info = plsc.get_sparse_core_info()
    nc, ns = info.num_cores, info.num_subcores
    assert nc * ns == NW
    mesh = plsc.VectorSubcoreMesh(core_axis_name="c", subcore_axis_name="s")

    @functools.partial(
        pl.kernel,
        mesh=mesh,
        compiler_params=pltpu.CompilerParams(needs_layout_passes=False),
        out_type=jax.ShapeDtypeStruct((BATCH + 16, 128), jnp.float32),
        scratch_types=[
            pltpu.VMEM((BATCH,), jnp.int32),
            pltpu.VMEM((BATCH + 16,), jnp.int32),
            pltpu.VMEM((BATCH + 16,), jnp.int32),
            pltpu.VMEM((2, EMBED, CHUNK), jnp.float32),
            pltpu.VMEM((NSLOT, 16, 128), jnp.float32),
            pltpu.VMEM((NSLOT, 16), jnp.int32),
            pltpu.SemaphoreType.DMA((2,)),
            pltpu.SemaphoreType.DMA((NSLOT,)),
        ],
    )
    def sweep_kernel(tabT_hbm, idx_hbm, out_hbm, idx_all, mem_pos, mem2_pos,
                     tile_buf, stage, pos_ring, ld_sems, sc_sem):
        wid = lax.axis_index("s") * nc + lax.axis_index("c")
        rlo = wid * RANGE
        rhi = rlo + RANGE
        slo = rlo - lax.rem(rlo, 128)
        iota = lax.iota(jnp.int32, 16)
        ones = jnp.ones((16,), jnp.int32)
        zeros = jnp.zeros((16,), jnp.int32)

        pltpu.sync_copy(idx_hbm, idx_all)

        def chunk_start(c):
            return jnp.minimum(slo + c * CHUNK, MAX_START)

        def issue(c):
            b = c % 2
            start = pl.multiple_of(chunk_start(c), 128)
            return pltpu.async_copy(
                tabT_hbm.at[:, pl.ds(start, CHUNK)],
                tile_buf.at[b],
                ld_sems.at[b],
            )

        cp0 = issue(0)

        def scan1(g, cnt):
            v = idx_all[pl.ds(g * 16, 16)]
            m = (v >= rlo) & (v < rhi)
            keys = jnp.where(m, zeros, ones)
            _, sv = plsc.sort_key_val(keys, g * 16 + iota)
            mem_pos[pl.ds(cnt, 16)] = sv
            return cnt + plsc.all_reduce_population_count(m)[0]

        mcnt = lax.fori_loop(0, BATCH // 16, scan1, jnp.int32(0), unroll=16)
        ng1 = (mcnt + 15) // 16

        def drain(slot):
            pltpu.make_async_copy(
                out_hbm.at[pl.ds(0, 16)], stage.at[slot], sc_sem.at[slot]
            ).wait()

        def process(tb, start, size, g_total):
            def scan2(g, cnt2):
                valid = (g * 16 + iota) < mcnt
                mp = mem_pos[pl.ds(g * 16, 16)]
                mp_safe = jnp.where(valid, mp, 0)
                mi = plsc.load_gather(idx_all, [mp_safe])
                m = valid & (mi >= start) & (mi < start + size)
                keys = jnp.where(m, zeros, ones)
                _, sv = plsc.sort_key_val(keys, mp_safe)
                mem2_pos[pl.ds(cnt2, 16)] = sv
                return cnt2 + plsc.all_reduce_population_count(m)[0]

            n2 = lax.fori_loop(0, ng1, scan2, jnp.int32(0))
            ng2 = (n2 + 15) // 16

            def proc(j, g_tot):
                slot = lax.rem(g_tot, NSLOT)
                valid = (j * 16 + iota) < n2
                mp = mem2_pos[pl.ds(j * 16, 16)]
                mp_safe = jnp.where(valid, mp, 0)
                mi = plsc.load_gather(idx_all, [mp_safe])
                lane = jnp.where(valid, mi - start, 0)
                pos = jnp.where(valid, mp_safe, DUMMY)

                @pl.when(g_tot >= NSLOT)
                def _():
                    drain(slot)

                slot_v = zeros + slot
                plsc.store_scatter(pos_ring, [slot_v, iota], pos)
                for k in range(EMBED):
                    kv = jnp.full((16,), k, jnp.int32)
                    vals = plsc.load_gather(tb, [kv, lane])
                    plsc.store_scatter(stage, [slot_v, iota, kv], vals)
                pltpu.async_copy(
                    stage.at[slot], out_hbm.at[pos_ring.at[slot]],
                    sc_sem.at[slot],
                )
                return g_tot + 1

            return lax.fori_loop(0, ng2, proc, g_total)

        g_total = jnp.int32(0)
        cp = cp0
        for c in range(NCH):
            cp_next = issue(c + 1) if c + 1 < NCH else None
            cp.wait()
            g_total = process(tile_buf.at[c % 2], chunk_start(c), CHUNK, g_total)
            cp = cp_next

        cps = [
            pltpu.async_copy(
                tabT_hbm.at[k, pl.ds(TAIL_START, TAIL)],
                tile_buf.at[0, k, pl.ds(0, TAIL)],
                ld_sems.at[0],
            )
            for k in range(EMBED)
        ]
        for tcp in cps:
            tcp.wait()
        g_total = process(tile_buf.at[0], jnp.int32(TAIL_START), TAIL, g_total)

        for s in range(NSLOT):
            @pl.when(g_total >= s + 1)
            def _():
                drain(s)

    return sweep_kernel


def _mlp_body(x_ref, w1_ref, b1_ref, w2_ref, b2_ref, w3_ref, b3_ref, o_ref):
    x = x_ref[:, :EMBED]
    h = jnp.dot(x, w1_ref[...], preferred_element_type=jnp.float32)
    h = jnp.maximum(h + b1_ref[...], 0.0)
    h = jnp.dot(h, w2_ref[...], preferred_element_type=jnp.float32)
    h = jnp.maximum(h + b2_ref[...], 0.0)
    o = jnp.dot(h, w3_ref[...], preferred_element_type=jnp.float32)
    o_ref[...] = jax.nn.sigmoid(o + b3_ref[...])


def _tc_mlp(x, W1, b1, W2, b2, W3, b3):
    nb = 8
    blk = BATCH // nb
    return pl.pallas_call(
        _mlp_body,
        grid=(nb,),
        in_specs=[
            pl.BlockSpec((blk, 128), lambda i: (i, 0)),
            pl.BlockSpec((EMBED, 32), lambda i: (0, 0)),
            pl.BlockSpec((1, 32), lambda i: (0, 0)),
            pl.BlockSpec((32, 16), lambda i: (0, 0)),
            pl.BlockSpec((1, 16), lambda i: (0, 0)),
            pl.BlockSpec((16, 1), lambda i: (0, 0)),
            pl.BlockSpec((1, 1), lambda i: (0, 0)),
        ],
        out_specs=pl.BlockSpec((blk, 1), lambda i: (i, 0)),
        out_shape=jax.ShapeDtypeStruct((BATCH, 1), jnp.float32),
    )(x, W1, b1, W2, b2, W3, b3)


def kernel(inputs, table, W1, b1, W2, b2, W3, b3):
    idx = inputs.astype(jnp.int32)
    rows = _make_sc_sweep()(table.T, idx)
    return _tc_mlp(
        rows,
        W1,
        b1.reshape(1, 32),
        W2,
        b2.reshape(1, 16),
        W3,
        b3.reshape(1, 1),
    )


# speedup vs baseline: 1.5199x; 1.5199x over previous
import functools

import jax
import jax.numpy as jnp
from jax import lax
from jax.experimental import pallas as pl
from jax.experimental.pallas import tpu as pltpu
from jax.experimental.pallas import tpu_sc as plsc

BATCH = 16384
EMBED = 16
NUM_ROWS = 1000000
NW = 32
RANGE = NUM_ROWS // NW
NCH = 8
CHUNK = 4096
TAIL_START = (NUM_ROWS // 128) * 128
TAIL = NUM_ROWS - TAIL_START
MAX_START = ((NUM_ROWS - CHUNK) // 128) * 128
NSLOT = 4
DUMMY = BATCH


@functools.lru_cache(maxsize=None)
def _make_sc_sweep():
    info = plsc.get_sparse_core_info()
    nc, ns = info.num_cores, info.num_subcores
    assert nc * ns == NW
    mesh = plsc.VectorSubcoreMesh(core_axis_name="c", subcore_axis_name="s")

    @functools.partial(
        pl.kernel,
        mesh=mesh,
        compiler_params=pltpu.CompilerParams(needs_layout_passes=False),
        out_type=jax.ShapeDtypeStruct((BATCH + 16, 128), jnp.float32),
        scratch_types=[
            pltpu.VMEM((BATCH,), jnp.int32),
            pltpu.VMEM((BATCH + 16,), jnp.int32),
            pltpu.VMEM((BATCH + 16,), jnp.int32),
            pltpu.VMEM((1, EMBED, CHUNK), jnp.float32),
            pltpu.VMEM((NSLOT, 16, 128), jnp.float32),
            pltpu.VMEM((NSLOT, 16), jnp.int32),
            pltpu.SemaphoreType.DMA((2,)),
            pltpu.SemaphoreType.DMA((NSLOT,)),
        ],
    )
    def sweep_kernel(tabT_hbm, idx_hbm, out_hbm, idx_all, mem_pos, mem2_pos,
                     tile_buf, stage, pos_ring, ld_sems, sc_sem):
        wid = lax.axis_index("s") * nc + lax.axis_index("c")
        rlo = wid * RANGE
        rhi = rlo + RANGE
        slo = rlo - lax.rem(rlo, 128)
        iota = lax.iota(jnp.int32, 16)
        ones = jnp.ones((16,), jnp.int32)
        zeros = jnp.zeros((16,), jnp.int32)

        pltpu.sync_copy(idx_hbm, idx_all)

        def chunk_start(c):
            return jnp.minimum(slo + c * CHUNK, MAX_START)

        def issue(c):
            start = pl.multiple_of(chunk_start(c), 128)
            return pltpu.async_copy(
                tabT_hbm.at[:, pl.ds(start, CHUNK)],
                tile_buf.at[0],
                ld_sems.at[0],
            )

        cp0 = issue(0)

        def scan1(g, cnt):
            v = idx_all[pl.ds(g * 16, 16)]
            m = (v >= rlo) & (v < rhi)
            keys = jnp.where(m, zeros, ones)
            _, sv = plsc.sort_key_val(keys, g * 16 + iota)
            mem_pos[pl.ds(cnt, 16)] = sv
            return cnt + plsc.all_reduce_population_count(m)[0]

        mcnt = lax.fori_loop(0, BATCH // 16, scan1, jnp.int32(0), unroll=16)
        ng1 = (mcnt + 15) // 16

        def drain(slot):
            pltpu.make_async_copy(
                out_hbm.at[pl.ds(0, 16)], stage.at[slot], sc_sem.at[slot]
            ).wait()

        def process(tb, start, size, g_total):
            def scan2(g, cnt2):
                valid = (g * 16 + iota) < mcnt
                mp = mem_pos[pl.ds(g * 16, 16)]
                mp_safe = jnp.where(valid, mp, 0)
                mi = plsc.load_gather(idx_all, [mp_safe])
                m = valid & (mi >= start) & (mi < start + size)
                keys = jnp.where(m, zeros, ones)
                _, sv = plsc.sort_key_val(keys, mp_safe)
                mem2_pos[pl.ds(cnt2, 16)] = sv
                return cnt2 + plsc.all_reduce_population_count(m)[0]

            n2 = lax.fori_loop(0, ng1, scan2, jnp.int32(0))
            ng2 = (n2 + 15) // 16

            def proc(j, g_tot):
                slot = lax.rem(g_tot, NSLOT)
                valid = (j * 16 + iota) < n2
                mp = mem2_pos[pl.ds(j * 16, 16)]
                mp_safe = jnp.where(valid, mp, 0)
                mi = plsc.load_gather(idx_all, [mp_safe])
                lane = jnp.where(valid, mi - start, 0)
                pos = jnp.where(valid, mp_safe, DUMMY)

                @pl.when(g_tot >= NSLOT)
                def _():
                    drain(slot)

                slot_v = zeros + slot
                plsc.store_scatter(pos_ring, [slot_v, iota], pos)
                for k in range(EMBED):
                    kv = jnp.full((16,), k, jnp.int32)
                    vals = plsc.load_gather(tb, [kv, lane])
                    plsc.store_scatter(stage, [slot_v, iota, kv], vals)
                pltpu.async_copy(
                    stage.at[slot], out_hbm.at[pos_ring.at[slot]],
                    sc_sem.at[slot],
                )
                return g_tot + 1

            return lax.fori_loop(0, ng2, proc, g_total)

        g_total = jnp.int32(0)
        cp0.wait()
        g_total = process(tile_buf.at[0], chunk_start(0), CHUNK, g_total)
        for c in range(1, NCH):
            issue(c).wait()
            g_total = process(tile_buf.at[0], chunk_start(c), CHUNK, g_total)

        cps = [
            pltpu.async_copy(
                tabT_hbm.at[k, pl.ds(TAIL_START, TAIL)],
                tile_buf.at[0, k, pl.ds(0, TAIL)],
                ld_sems.at[0],
            )
            for k in range(EMBED)
        ]
        for tcp in cps:
            tcp.wait()
        g_total = process(tile_buf.at[0], jnp.int32(TAIL_START), TAIL, g_total)

        for s in range(NSLOT):
            @pl.when(g_total >= s + 1)
            def _():
                drain(s)

    return sweep_kernel


def _mlp_body(x_ref, w1_ref, b1_ref, w2_ref, b2_ref, w3_ref, b3_ref, o_ref):
    x = x_ref[:, :EMBED]
    h = jnp.dot(x, w1_ref[...], preferred_element_type=jnp.float32)
    h = jnp.maximum(h + b1_ref[...], 0.0)
    h = jnp.dot(h, w2_ref[...], preferred_element_type=jnp.float32)
    h = jnp.maximum(h + b2_ref[...], 0.0)
    o = jnp.dot(h, w3_ref[...], preferred_element_type=jnp.float32)
    o_ref[...] = jax.nn.sigmoid(o + b3_ref[...])


def _tc_mlp(x, W1, b1, W2, b2, W3, b3):
    nb = 8
    blk = BATCH // nb
    return pl.pallas_call(
        _mlp_body,
        grid=(nb,),
        in_specs=[
            pl.BlockSpec((blk, 128), lambda i: (i, 0)),
            pl.BlockSpec((EMBED, 32), lambda i: (0, 0)),
            pl.BlockSpec((1, 32), lambda i: (0, 0)),
            pl.BlockSpec((32, 16), lambda i: (0, 0)),
            pl.BlockSpec((1, 16), lambda i: (0, 0)),
            pl.BlockSpec((16, 1), lambda i: (0, 0)),
            pl.BlockSpec((1, 1), lambda i: (0, 0)),
        ],
        out_specs=pl.BlockSpec((blk, 1), lambda i: (i, 0)),
        out_shape=jax.ShapeDtypeStruct((BATCH, 1), jnp.float32),
    )(x, W1, b1, W2, b2, W3, b3)


def kernel(inputs, table, W1, b1, W2, b2, W3, b3):
    idx = inputs.astype(jnp.int32)
    rows = _make_sc_sweep()(table.T, idx)
    return _tc_mlp(
        rows,
        W1,
        b1.reshape(1, 32),
        W2,
        b2.reshape(1, 16),
        W3,
        b3.reshape(1, 1),
    )
